# baseline (speedup 1.0000x reference)
import functools

import jax
import jax.numpy as jnp
from jax import lax
from jax.experimental import pallas as pl
from jax.experimental.pallas import tpu as pltpu
from jax.experimental.pallas import tpu_sc as plsc

N = 10000
E = 320000
D = 128
H = 128
DH = D // 2

NC = 2
NS = 16
K = 128
ER = E // K
TROWS = ER // NS
LE = ER - NS * TROWS
SEG = 3
SROWS = TROWS // SEG
DROWS = ER // (NC * NS)
DLE = ER - NC * NS * DROWS

RPT = 624
RTAIL = N - NS * RPT

_MESH = plsc.VectorSubcoreMesh(core_axis_name="c", subcore_axis_name="s")
_SC_PARAMS = pltpu.CompilerParams(use_tc_tiling_on_sc=False)


@functools.partial(
    pl.kernel,
    out_type=jax.ShapeDtypeStruct((NC * 2 * N,), jnp.float32),
    mesh=_MESH,
    scratch_types=[
        pltpu.VMEM((DROWS, K), jnp.int32),
        pltpu.VMEM((DROWS, K), jnp.int32),
        pltpu.VMEM((K,), jnp.float32),
        pltpu.VMEM((RPT,), jnp.float32),
        pltpu.VMEM_SHARED((N,), jnp.float32),
        pltpu.VMEM_SHARED((N,), jnp.float32),
        pltpu.SemaphoreType.DMA,
    ],
    compiler_params=_SC_PARAMS,
)
def _deg_kernel(edge_hbm, zeros1_hbm, ones_hbm, out_hbm,
                src_all, dst_all, ones_v, stage_v, acc_src, acc_dst, sem):
    cid = lax.axis_index("c")
    sid = lax.axis_index("s")
    wid = cid * NS + sid

    pltpu.sync_copy(edge_hbm.at[pl.ds(wid * DROWS, DROWS)], src_all)
    pltpu.sync_copy(edge_hbm.at[pl.ds(ER + wid * DROWS, DROWS)], dst_all)

    pltpu.sync_copy(zeros1_hbm, stage_v)
    pltpu.sync_copy(stage_v, acc_src.at[pl.ds(sid * RPT, RPT)])
    pltpu.sync_copy(stage_v, acc_dst.at[pl.ds(sid * RPT, RPT)])

    @pl.when(sid == NS - 1)
    def _zero_tail():
        pltpu.sync_copy(stage_v.at[pl.ds(0, RTAIL)],
                        acc_src.at[pl.ds(NS * RPT, RTAIL)])
        pltpu.sync_copy(stage_v.at[pl.ds(0, RTAIL)],
                        acc_dst.at[pl.ds(NS * RPT, RTAIL)])

    pltpu.sync_copy(ones_hbm, ones_v)
    plsc.subcore_barrier()

    def body(i, carry):
        pltpu.sync_copy(ones_v, acc_src.at[src_all.at[i]], add=True)
        pltpu.sync_copy(ones_v, acc_dst.at[dst_all.at[i]], add=True)
        return carry

    lax.fori_loop(0, DROWS, body, 0)

    @pl.when(wid < DLE)
    def _extra():
        row = NC * NS * DROWS + wid
        pltpu.sync_copy(edge_hbm.at[pl.ds(row, 1)], src_all.at[pl.ds(0, 1)])
        pltpu.sync_copy(edge_hbm.at[pl.ds(ER + row, 1)],
                        dst_all.at[pl.ds(0, 1)])
        pltpu.sync_copy(ones_v, acc_src.at[src_all.at[0]], add=True)
        pltpu.sync_copy(ones_v, acc_dst.at[dst_all.at[0]], add=True)

    plsc.subcore_barrier()

    obase = cid * 2 * N
    pltpu.sync_copy(acc_src.at[pl.ds(sid * RPT, RPT)], stage_v)
    pltpu.sync_copy(stage_v, out_hbm.at[pl.ds(obase + sid * RPT, RPT)])
    pltpu.sync_copy(acc_dst.at[pl.ds(sid * RPT, RPT)], stage_v)
    pltpu.sync_copy(stage_v, out_hbm.at[pl.ds(obase + N + sid * RPT, RPT)])

    @pl.when(sid == NS - 1)
    def _write_tail():
        pltpu.sync_copy(acc_src.at[pl.ds(NS * RPT, RTAIL)],
                        stage_v.at[pl.ds(0, RTAIL)])
        pltpu.sync_copy(stage_v.at[pl.ds(0, RTAIL)],
                        out_hbm.at[pl.ds(obase + NS * RPT, RTAIL)])
        pltpu.sync_copy(acc_dst.at[pl.ds(NS * RPT, RTAIL)],
                        stage_v.at[pl.ds(0, RTAIL)])
        pltpu.sync_copy(stage_v.at[pl.ds(0, RTAIL)],
                        out_hbm.at[pl.ds(obase + N + NS * RPT, RTAIL)])


@functools.partial(
    pl.kernel,
    out_type=jax.ShapeDtypeStruct((N, NC, DH), jnp.float32),
    mesh=_MESH,
    scratch_types=[
        pltpu.VMEM((SROWS, K), jnp.int32),
        pltpu.VMEM((SROWS, K), jnp.int32),
        pltpu.VMEM((K, DH), jnp.float32),
        pltpu.VMEM((K, DH), jnp.float32),
        pltpu.VMEM((K, DH), jnp.float32),
        pltpu.VMEM((K, DH), jnp.float32),
        pltpu.VMEM_SHARED((N, DH), jnp.float32),
        pltpu.SemaphoreType.DMA,
        pltpu.SemaphoreType.DMA,
        pltpu.SemaphoreType.DMA,
        pltpu.SemaphoreType.DMA,
        pltpu.SemaphoreType.DMA,
        pltpu.SemaphoreType.DMA,
        pltpu.SemaphoreType.DMA,
        pltpu.SemaphoreType.DMA,
    ],
    compiler_params=_SC_PARAMS,
)
def _agg_kernel(feat_hbm, edge_hbm, zeros2_hbm, out_hbm,
                src_seg, dst_seg, rows0, rows1, rows2, rows3, acc_sh,
                gs0, gs1, gs2, gs3, ss0, ss1, ss2, ss3):
    cid = lax.axis_index("c")
    sid = lax.axis_index("s")

    pltpu.sync_copy(zeros2_hbm, rows0)
    for j in range(RPT // K):
        pltpu.sync_copy(rows0, acc_sh.at[pl.ds(sid * RPT + j * K, K)])
    pltpu.sync_copy(rows0.at[pl.ds(0, RPT % K)],
                    acc_sh.at[pl.ds(sid * RPT + (RPT // K) * K, RPT % K)])

    @pl.when(sid == NS - 1)
    def _zero_tail():
        pltpu.sync_copy(rows0.at[pl.ds(0, RTAIL)],
                        acc_sh.at[pl.ds(NS * RPT, RTAIL)])

    plsc.subcore_barrier()

    bufs = (rows0, rows1, rows2, rows3)
    gsems = (gs0, gs1, gs2, gs3)
    ssems = (ss0, ss1, ss2, ss3)

    def wait_gather(i, b):
        pltpu.make_async_copy(feat_hbm.at[src_seg.at[i]], bufs[b],
                              gsems[b]).wait()

    def wait_scatter(b):
        pltpu.make_async_copy(bufs[b], acc_sh.at[dst_seg.at[0]],
                              ssems[b]).wait()

    def xform_src(i):
        for j in range(K // 16):
            sl = pl.ds(j * 16, 16)
            src_seg[i, sl] = src_seg[i, sl] * 2 + cid

    def seg_body(s, carry):
        base_row = sid * TROWS + s * SROWS
        pltpu.sync_copy(edge_hbm.at[pl.ds(base_row, SROWS)], src_seg)
        pltpu.sync_copy(edge_hbm.at[pl.ds(ER + base_row, SROWS)], dst_seg)

        def xbody(i, c):
            xform_src(i)
            return c

        lax.fori_loop(0, SROWS, xbody, 0)

        pltpu.async_copy(feat_hbm.at[src_seg.at[0]], rows0, gs0)
        pltpu.async_copy(feat_hbm.at[src_seg.at[1]], rows1, gs1)

        def body(g, c):
            for b in range(4):
                i = g * 4 + b
                wait_gather(i, b)
                pltpu.async_copy(bufs[b], acc_sh.at[dst_seg.at[i]],
                                 ssems[b], add=True)
                b2 = (b + 2) % 4

                @pl.when(i + 2 < SROWS)
                def _prefetch():
                    @pl.when(i >= 2)
                    def _recycle():
                        wait_scatter(b2)

                    pltpu.async_copy(feat_hbm.at[src_seg.at[i + 2]], bufs[b2],
                                     gsems[b2])
            return c

        lax.fori_loop(0, SROWS // 4, body, 0)
        for b in range(4):
            wait_scatter(b)
        return carry

    lax.fori_loop(0, SEG, seg_body, 0)

    @pl.when(sid < LE)
    def _extra():
        row = NS * TROWS + sid
        pltpu.sync_copy(edge_hbm.at[pl.ds(row, 1)], src_seg.at[pl.ds(0, 1)])
        pltpu.sync_copy(edge_hbm.at[pl.ds(ER + row, 1)],
                        dst_seg.at[pl.ds(0, 1)])
        xform_src(0)
        pltpu.async_copy(feat_hbm.at[src_seg.at[0]], rows0, gs0)
        wait_gather(0, 0)
        pltpu.sync_copy(rows0, acc_sh.at[dst_seg.at[0]], add=True)

    plsc.subcore_barrier()

    for j in range(RPT // K):
        pltpu.sync_copy(acc_sh.at[pl.ds(sid * RPT + j * K, K)], rows0)
        pltpu.sync_copy(rows0, out_hbm.at[pl.ds(sid * RPT + j * K, K), cid])
    last = RPT % K
    pltpu.sync_copy(acc_sh.at[pl.ds(sid * RPT + (RPT // K) * K, last)],
                    rows0.at[pl.ds(0, last)])
    pltpu.sync_copy(rows0.at[pl.ds(0, last)],
                    out_hbm.at[pl.ds(sid * RPT + (RPT // K) * K, last), cid])

    @pl.when(sid == NS - 1)
    def _write_tail():
        pltpu.sync_copy(acc_sh.at[pl.ds(NS * RPT, RTAIL)],
                        rows1.at[pl.ds(0, RTAIL)])
        pltpu.sync_copy(rows1.at[pl.ds(0, RTAIL)],
                        out_hbm.at[pl.ds(NS * RPT, RTAIL), cid])


RB = 1000
GRID = N // RB


def _norm_body(deg_ref, x_ref, xp_ref, ns_ref, nd_ref):
    deg = deg_ref[...]
    dsrc = deg[:, 0:1] + deg[:, 2:3]
    ddst = deg[:, 1:2] + deg[:, 3:4]
    ns = lax.rsqrt(jnp.where(dsrc > 0, dsrc, 1.0))
    nd = lax.rsqrt(jnp.where(ddst > 0, ddst, 1.0))
    xp_ref[...] = x_ref[...] * ns
    ns_ref[...] = ns
    nd_ref[...] = nd


def _layer1_body(a_ref, w_ref, b_ref, nd_ref, ns_ref, out_ref):
    h = jnp.dot(a_ref[...], w_ref[...], preferred_element_type=jnp.float32,
                precision=lax.Precision.HIGHEST)
    h = jnp.maximum(h * nd_ref[...] + b_ref[...], 0.0)
    out_ref[...] = h * ns_ref[...]


def _final_body(a_ref, w_ref, b_ref, nd_ref,
                wc1_ref, bc1_ref, wc2_ref, bc2_ref, wc3_ref, bc3_ref,
                out_ref, acc_ref):
    i = pl.program_id(0)

    @pl.when(i == 0)
    def _init():
        acc_ref[...] = jnp.zeros_like(acc_ref)

    h = jnp.dot(a_ref[...], w_ref[...], preferred_element_type=jnp.float32,
                precision=lax.Precision.HIGHEST)
    h = jnp.maximum(h * nd_ref[...] + b_ref[...], 0.0)
    acc_ref[...] += jnp.sum(h, axis=0, keepdims=True)

    @pl.when(i == pl.num_programs(0) - 1)
    def _head():
        hg = acc_ref[...] * (1.0 / N)
        o = jnp.dot(hg, wc1_ref[...], preferred_element_type=jnp.float32,
                    precision=lax.Precision.HIGHEST)
        o = jnp.maximum(o + bc1_ref[...], 0.0)
        o = jnp.dot(o, wc2_ref[...], preferred_element_type=jnp.float32,
                    precision=lax.Precision.HIGHEST)
        o = jnp.maximum(o + bc2_ref[...], 0.0)
        out_ref[...] = (jnp.dot(o, wc3_ref[...],
                                preferred_element_type=jnp.float32,
                                precision=lax.Precision.HIGHEST)
                        + bc3_ref[...])


def kernel(x, edge_index, W1, b1, W2, b2, Wc1, bc1, Wc2, bc2, Wc3, bc3):
    edge2d = edge_index.astype(jnp.int32).reshape(2 * ER, K)
    zeros1 = jnp.zeros((RPT,), jnp.float32)
    zeros2 = jnp.zeros((DH, D), jnp.float32).reshape(K, DH)
    ones_k = jnp.ones((K,), jnp.float32)

    deg = _deg_kernel(edge2d, zeros1, ones_k)
    degT = deg.reshape(2 * NC, N).T

    xp, nsrc, ndst = pl.pallas_call(
        _norm_body,
        grid=(GRID,),
        in_specs=[
            pl.BlockSpec((RB, 2 * NC), lambda i: (i, 0)),
            pl.BlockSpec((RB, D), lambda i: (i, 0)),
        ],
        out_specs=[
            pl.BlockSpec((RB, D), lambda i: (i, 0)),
            pl.BlockSpec((RB, 1), lambda i: (i, 0)),
            pl.BlockSpec((RB, 1), lambda i: (i, 0)),
        ],
        out_shape=[
            jax.ShapeDtypeStruct((N, D), jnp.float32),
            jax.ShapeDtypeStruct((N, 1), jnp.float32),
            jax.ShapeDtypeStruct((N, 1), jnp.float32),
        ],
    )(degT, x)

    agg1 = _agg_kernel(xp.reshape(2 * N, DH), edge2d, zeros2)
    agg1c = agg1.reshape(N, D)

    h1p = pl.pallas_call(
        _layer1_body,
        grid=(GRID,),
        in_specs=[
            pl.BlockSpec((RB, D), lambda i: (i, 0)),
            pl.BlockSpec((D, H), lambda i: (0, 0)),
            pl.BlockSpec((1, H), lambda i: (0, 0)),
            pl.BlockSpec((RB, 1), lambda i: (i, 0)),
            pl.BlockSpec((RB, 1), lambda i: (i, 0)),
        ],
        out_specs=pl.BlockSpec((RB, H), lambda i: (i, 0)),
        out_shape=jax.ShapeDtypeStruct((N, H), jnp.float32),
    )(agg1c, W1, b1.reshape(1, H), ndst, nsrc)

    agg2 = _agg_kernel(h1p.reshape(2 * N, DH), edge2d, zeros2)
    agg2c = agg2.reshape(N, H)

    out = pl.pallas_call(
        _final_body,
        grid=(GRID,),
        in_specs=[
            pl.BlockSpec((RB, H), lambda i: (i, 0)),
            pl.BlockSpec((H, H), lambda i: (0, 0)),
            pl.BlockSpec((1, H), lambda i: (0, 0)),
            pl.BlockSpec((RB, 1), lambda i: (i, 0)),
            pl.BlockSpec((H, H), lambda i: (0, 0)),
            pl.BlockSpec((1, H), lambda i: (0, 0)),
            pl.BlockSpec((H, H), lambda i: (0, 0)),
            pl.BlockSpec((1, H), lambda i: (0, 0)),
            pl.BlockSpec((H, 1), lambda i: (0, 0)),
            pl.BlockSpec((1, 1), lambda i: (0, 0)),
        ],
        out_specs=pl.BlockSpec((1, 1), lambda i: (0, 0)),
        out_shape=jax.ShapeDtypeStruct((1, 1), jnp.float32),
        scratch_shapes=[pltpu.VMEM((1, H), jnp.float32)],
    )(agg2c, W2, b2.reshape(1, H), ndst,
      Wc1, bc1.reshape(1, H), Wc2, bc2.reshape(1, H),
      Wc3, bc3.reshape(1, 1))

    return out


# speedup vs baseline: 1.0016x; 1.0016x over previous
import functools

import jax
import jax.numpy as jnp
from jax import lax
from jax.experimental import pallas as pl
from jax.experimental.pallas import tpu as pltpu
from jax.experimental.pallas import tpu_sc as plsc

N = 10000
E = 320000
D = 128
H = 128
DH = D // 2

NC = 2
NS = 16
K = 128
ER = E // K
TROWS = ER // NS
LE = ER - NS * TROWS
SEG = 3
SROWS = TROWS // SEG
DROWS = ER // (NC * NS)
DLE = ER - NC * NS * DROWS

RPT = 624
RTAIL = N - NS * RPT

_MESH = plsc.VectorSubcoreMesh(core_axis_name="c", subcore_axis_name="s")
_SC_PARAMS = pltpu.CompilerParams(use_tc_tiling_on_sc=False)


@functools.partial(
    pl.kernel,
    out_type=jax.ShapeDtypeStruct((NC * 2 * N,), jnp.float32),
    mesh=_MESH,
    scratch_types=[
        pltpu.VMEM((DROWS, K), jnp.int32),
        pltpu.VMEM((DROWS, K), jnp.int32),
        pltpu.VMEM((K,), jnp.float32),
        pltpu.VMEM((RPT,), jnp.float32),
        pltpu.VMEM_SHARED((N,), jnp.float32),
        pltpu.VMEM_SHARED((N,), jnp.float32),
        pltpu.SemaphoreType.DMA,
    ],
    compiler_params=_SC_PARAMS,
)
def _deg_kernel(edge_hbm, zeros1_hbm, ones_hbm, out_hbm,
                src_all, dst_all, ones_v, stage_v, acc_src, acc_dst, sem):
    cid = lax.axis_index("c")
    sid = lax.axis_index("s")
    wid = cid * NS + sid

    pltpu.sync_copy(edge_hbm.at[pl.ds(wid * DROWS, DROWS)], src_all)
    pltpu.sync_copy(edge_hbm.at[pl.ds(ER + wid * DROWS, DROWS)], dst_all)

    pltpu.sync_copy(zeros1_hbm, stage_v)
    pltpu.sync_copy(stage_v, acc_src.at[pl.ds(sid * RPT, RPT)])
    pltpu.sync_copy(stage_v, acc_dst.at[pl.ds(sid * RPT, RPT)])

    @pl.when(sid == NS - 1)
    def _zero_tail():
        pltpu.sync_copy(stage_v.at[pl.ds(0, RTAIL)],
                        acc_src.at[pl.ds(NS * RPT, RTAIL)])
        pltpu.sync_copy(stage_v.at[pl.ds(0, RTAIL)],
                        acc_dst.at[pl.ds(NS * RPT, RTAIL)])

    pltpu.sync_copy(ones_hbm, ones_v)
    plsc.subcore_barrier()

    def body(i, carry):
        pltpu.sync_copy(ones_v, acc_src.at[src_all.at[i]], add=True)
        pltpu.sync_copy(ones_v, acc_dst.at[dst_all.at[i]], add=True)
        return carry

    lax.fori_loop(0, DROWS, body, 0)

    @pl.when(wid < DLE)
    def _extra():
        row = NC * NS * DROWS + wid
        pltpu.sync_copy(edge_hbm.at[pl.ds(row, 1)], src_all.at[pl.ds(0, 1)])
        pltpu.sync_copy(edge_hbm.at[pl.ds(ER + row, 1)],
                        dst_all.at[pl.ds(0, 1)])
        pltpu.sync_copy(ones_v, acc_src.at[src_all.at[0]], add=True)
        pltpu.sync_copy(ones_v, acc_dst.at[dst_all.at[0]], add=True)

    plsc.subcore_barrier()

    obase = cid * 2 * N
    pltpu.sync_copy(acc_src.at[pl.ds(sid * RPT, RPT)], stage_v)
    pltpu.sync_copy(stage_v, out_hbm.at[pl.ds(obase + sid * RPT, RPT)])
    pltpu.sync_copy(acc_dst.at[pl.ds(sid * RPT, RPT)], stage_v)
    pltpu.sync_copy(stage_v, out_hbm.at[pl.ds(obase + N + sid * RPT, RPT)])

    @pl.when(sid == NS - 1)
    def _write_tail():
        pltpu.sync_copy(acc_src.at[pl.ds(NS * RPT, RTAIL)],
                        stage_v.at[pl.ds(0, RTAIL)])
        pltpu.sync_copy(stage_v.at[pl.ds(0, RTAIL)],
                        out_hbm.at[pl.ds(obase + NS * RPT, RTAIL)])
        pltpu.sync_copy(acc_dst.at[pl.ds(NS * RPT, RTAIL)],
                        stage_v.at[pl.ds(0, RTAIL)])
        pltpu.sync_copy(stage_v.at[pl.ds(0, RTAIL)],
                        out_hbm.at[pl.ds(obase + N + NS * RPT, RTAIL)])


@functools.partial(
    pl.kernel,
    out_type=jax.ShapeDtypeStruct((N, NC, DH), jnp.float32),
    mesh=_MESH,
    scratch_types=[
        pltpu.VMEM((SROWS, K), jnp.int32),
        pltpu.VMEM((SROWS, K), jnp.int32),
        pltpu.VMEM((K, DH), jnp.float32),
        pltpu.VMEM((K, DH), jnp.float32),
        pltpu.VMEM((K, DH), jnp.float32),
        pltpu.VMEM((K, DH), jnp.float32),
        pltpu.VMEM_SHARED((N, DH), jnp.float32),
        pltpu.SemaphoreType.DMA,
        pltpu.SemaphoreType.DMA,
        pltpu.SemaphoreType.DMA,
        pltpu.SemaphoreType.DMA,
        pltpu.SemaphoreType.DMA,
        pltpu.SemaphoreType.DMA,
        pltpu.SemaphoreType.DMA,
        pltpu.SemaphoreType.DMA,
    ],
    compiler_params=_SC_PARAMS,
)
def _agg_kernel(feat_hbm, edge_hbm, zeros2_hbm, out_hbm,
                src_seg, dst_seg, rows0, rows1, rows2, rows3, acc_sh,
                gs0, gs1, gs2, gs3, ss0, ss1, ss2, ss3):
    cid = lax.axis_index("c")
    sid = lax.axis_index("s")

    pltpu.sync_copy(zeros2_hbm, rows0)
    for j in range(RPT // K):
        pltpu.sync_copy(rows0, acc_sh.at[pl.ds(sid * RPT + j * K, K)])
    pltpu.sync_copy(rows0.at[pl.ds(0, RPT % K)],
                    acc_sh.at[pl.ds(sid * RPT + (RPT // K) * K, RPT % K)])

    @pl.when(sid == NS - 1)
    def _zero_tail():
        pltpu.sync_copy(rows0.at[pl.ds(0, RTAIL)],
                        acc_sh.at[pl.ds(NS * RPT, RTAIL)])

    plsc.subcore_barrier()

    bufs = (rows0, rows1, rows2, rows3)
    gsems = (gs0, gs1, gs2, gs3)
    ssems = (ss0, ss1, ss2, ss3)

    def wait_gather(i, b):
        pltpu.make_async_copy(feat_hbm.at[src_seg.at[i]], bufs[b],
                              gsems[b]).wait()

    def wait_scatter(b):
        pltpu.make_async_copy(bufs[b], acc_sh.at[dst_seg.at[0]],
                              ssems[b]).wait()

    def xform_src(i):
        for j in range(K // 16):
            sl = pl.ds(j * 16, 16)
            src_seg[i, sl] = src_seg[i, sl] * 2 + cid

    def seg_body(s, carry):
        base_row = sid * TROWS + s * SROWS
        pltpu.sync_copy(edge_hbm.at[pl.ds(base_row, SROWS)], src_seg)
        pltpu.sync_copy(edge_hbm.at[pl.ds(ER + base_row, SROWS)], dst_seg)

        def xbody(i, c):
            xform_src(i)
            return c

        lax.fori_loop(0, SROWS, xbody, 0)

        pltpu.async_copy(feat_hbm.at[src_seg.at[0]], rows0, gs0)
        pltpu.async_copy(feat_hbm.at[src_seg.at[1]], rows1, gs1)

        def body(g, c):
            for b in range(4):
                i = g * 4 + b
                wait_gather(i, b)
                pltpu.async_copy(bufs[b], acc_sh.at[dst_seg.at[i]],
                                 ssems[b], add=True)
                b2 = (b + 2) % 4

                @pl.when(i + 2 < SROWS)
                def _prefetch():
                    @pl.when(i >= 2)
                    def _recycle():
                        wait_scatter(b2)

                    pltpu.async_copy(feat_hbm.at[src_seg.at[i + 2]], bufs[b2],
                                     gsems[b2])
            return c

        lax.fori_loop(0, SROWS // 4, body, 0)
        for b in range(4):
            wait_scatter(b)
        return carry

    lax.fori_loop(0, SEG, seg_body, 0)

    @pl.when(sid < LE)
    def _extra():
        row = NS * TROWS + sid
        pltpu.sync_copy(edge_hbm.at[pl.ds(row, 1)], src_seg.at[pl.ds(0, 1)])
        pltpu.sync_copy(edge_hbm.at[pl.ds(ER + row, 1)],
                        dst_seg.at[pl.ds(0, 1)])
        xform_src(0)
        pltpu.async_copy(feat_hbm.at[src_seg.at[0]], rows0, gs0)
        wait_gather(0, 0)
        pltpu.sync_copy(rows0, acc_sh.at[dst_seg.at[0]], add=True)

    plsc.subcore_barrier()

    for j in range(RPT // K):
        pltpu.sync_copy(acc_sh.at[pl.ds(sid * RPT + j * K, K)], rows0)
        pltpu.sync_copy(rows0, out_hbm.at[pl.ds(sid * RPT + j * K, K), cid])
    last = RPT % K
    pltpu.sync_copy(acc_sh.at[pl.ds(sid * RPT + (RPT // K) * K, last)],
                    rows0.at[pl.ds(0, last)])
    pltpu.sync_copy(rows0.at[pl.ds(0, last)],
                    out_hbm.at[pl.ds(sid * RPT + (RPT // K) * K, last), cid])

    @pl.when(sid == NS - 1)
    def _write_tail():
        pltpu.sync_copy(acc_sh.at[pl.ds(NS * RPT, RTAIL)],
                        rows1.at[pl.ds(0, RTAIL)])
        pltpu.sync_copy(rows1.at[pl.ds(0, RTAIL)],
                        out_hbm.at[pl.ds(NS * RPT, RTAIL), cid])


RB = 1000
GRID = N // RB


def _norm_body(deg_ref, x_ref, w_ref, xw_ref, ns_ref, nd_ref):
    deg = deg_ref[...]
    dsrc = deg[:, 0:1] + deg[:, 2:3]
    ddst = deg[:, 1:2] + deg[:, 3:4]
    ns = lax.rsqrt(jnp.where(dsrc > 0, dsrc, 1.0))
    nd = lax.rsqrt(jnp.where(ddst > 0, ddst, 1.0))
    xw_ref[...] = jnp.dot(x_ref[...] * ns, w_ref[...],
                          preferred_element_type=jnp.float32, precision=lax.Precision.HIGHEST)
    ns_ref[...] = ns
    nd_ref[...] = nd


def _layer1_body(a_ref, w_ref, b_ref, nd_ref, ns_ref, out_ref):
    h = jnp.maximum(a_ref[...] * nd_ref[...] + b_ref[...], 0.0)
    out_ref[...] = jnp.dot(h * ns_ref[...], w_ref[...],
                           preferred_element_type=jnp.float32, precision=lax.Precision.HIGHEST)


def _final_body(a_ref, b_ref, nd_ref,
                wc1_ref, bc1_ref, wc2_ref, bc2_ref, wc3_ref, bc3_ref,
                out_ref, acc_ref):
    i = pl.program_id(0)

    @pl.when(i == 0)
    def _init():
        acc_ref[...] = jnp.zeros_like(acc_ref)

    h = jnp.maximum(a_ref[...] * nd_ref[...] + b_ref[...], 0.0)
    acc_ref[...] += jnp.sum(h, axis=0, keepdims=True)

    @pl.when(i == pl.num_programs(0) - 1)
    def _head():
        hg = acc_ref[...] * (1.0 / N)
        o = jnp.dot(hg, wc1_ref[...], preferred_element_type=jnp.float32, precision=lax.Precision.HIGHEST)
        o = jnp.maximum(o + bc1_ref[...], 0.0)
        o = jnp.dot(o, wc2_ref[...], preferred_element_type=jnp.float32, precision=lax.Precision.HIGHEST)
        o = jnp.maximum(o + bc2_ref[...], 0.0)
        out_ref[...] = (jnp.dot(o, wc3_ref[...],
                                preferred_element_type=jnp.float32, precision=lax.Precision.HIGHEST)
                        + bc3_ref[...])


def kernel(x, edge_index, W1, b1, W2, b2, Wc1, bc1, Wc2, bc2, Wc3, bc3):
    edge2d = edge_index.astype(jnp.int32).reshape(2 * ER, K)
    zeros1 = jnp.zeros((RPT,), jnp.float32)
    zeros2 = jnp.zeros((DH, D), jnp.float32).reshape(K, DH)
    ones_k = jnp.ones((K,), jnp.float32)

    deg = _deg_kernel(edge2d, zeros1, ones_k)
    degT = deg.reshape(2 * NC, N).T

    xw, nsrc, ndst = pl.pallas_call(
        _norm_body,
        grid=(GRID,),
        in_specs=[
            pl.BlockSpec((RB, 2 * NC), lambda i: (i, 0)),
            pl.BlockSpec((RB, D), lambda i: (i, 0)),
            pl.BlockSpec((D, H), lambda i: (0, 0)),
        ],
        out_specs=[
            pl.BlockSpec((RB, H), lambda i: (i, 0)),
            pl.BlockSpec((RB, 1), lambda i: (i, 0)),
            pl.BlockSpec((RB, 1), lambda i: (i, 0)),
        ],
        out_shape=[
            jax.ShapeDtypeStruct((N, H), jnp.float32),
            jax.ShapeDtypeStruct((N, 1), jnp.float32),
            jax.ShapeDtypeStruct((N, 1), jnp.float32),
        ],
    )(degT, x, W1)

    agg1 = _agg_kernel(xw.reshape(2 * N, DH), edge2d, zeros2)
    agg1c = agg1.reshape(N, H)

    h1w = pl.pallas_call(
        _layer1_body,
        grid=(GRID,),
        in_specs=[
            pl.BlockSpec((RB, H), lambda i: (i, 0)),
            pl.BlockSpec((H, H), lambda i: (0, 0)),
            pl.BlockSpec((1, H), lambda i: (0, 0)),
            pl.BlockSpec((RB, 1), lambda i: (i, 0)),
            pl.BlockSpec((RB, 1), lambda i: (i, 0)),
        ],
        out_specs=pl.BlockSpec((RB, H), lambda i: (i, 0)),
        out_shape=jax.ShapeDtypeStruct((N, H), jnp.float32),
    )(agg1c, W2, b1.reshape(1, H), ndst, nsrc)

    agg2 = _agg_kernel(h1w.reshape(2 * N, DH), edge2d, zeros2)
    agg2c = agg2.reshape(N, H)

    out = pl.pallas_call(
        _final_body,
        grid=(GRID,),
        in_specs=[
            pl.BlockSpec((RB, H), lambda i: (i, 0)),
            pl.BlockSpec((1, H), lambda i: (0, 0)),
            pl.BlockSpec((RB, 1), lambda i: (i, 0)),
            pl.BlockSpec((H, H), lambda i: (0, 0)),
            pl.BlockSpec((1, H), lambda i: (0, 0)),
            pl.BlockSpec((H, H), lambda i: (0, 0)),
            pl.BlockSpec((1, H), lambda i: (0, 0)),
            pl.BlockSpec((H, 1), lambda i: (0, 0)),
            pl.BlockSpec((1, 1), lambda i: (0, 0)),
        ],
        out_specs=pl.BlockSpec((1, 1), lambda i: (0, 0)),
        out_shape=jax.ShapeDtypeStruct((1, 1), jnp.float32),
        scratch_shapes=[pltpu.VMEM((1, H), jnp.float32)],
    )(agg2c, b2.reshape(1, H), ndst,
      Wc1, bc1.reshape(1, H), Wc2, bc2.reshape(1, H),
      Wc3, bc3.reshape(1, 1))

    return out
